# Initial kernel scaffold; baseline (speedup 1.0000x reference)
import jax
import jax.numpy as jnp
from jax.experimental import pallas as pl


def kernel(midis_out):
    raise NotImplementedError("write your pallas kernel here")


# speedup vs baseline: 1.8526x; 1.8526x over previous
import jax
import jax.numpy as jnp
from jax.experimental import pallas as pl

_LST = 64
_BISECT_ITERS = 26


def _bisect_tau(x, axis):
    zmax = jnp.max(x, axis=axis, keepdims=True)
    lo = zmax - 1.0
    hi = zmax

    def body(_, carry):
        lo, hi = carry
        mid = 0.5 * (lo + hi)
        s = jnp.sum(jnp.maximum(x - mid, 0.0), axis=axis, keepdims=True)
        pred = s >= 1.0
        lo = jnp.where(pred, mid, lo)
        hi = jnp.where(pred, hi, mid)
        return lo, hi

    lo, hi = jax.lax.fori_loop(0, _BISECT_ITERS, body, (lo, hi))
    return 0.5 * (lo + hi)


def _inst_kernel(x_ref, o_ref):
    x = x_ref[0]
    tau = _bisect_tau(x, axis=0)
    o_ref[0] = jnp.maximum(x - tau, 0.0)


def _time_kernel(x_ref, oi_ref, o_ref):
    x = x_ref[...]
    tau = _bisect_tau(x, axis=1)
    o_ref[...] = jnp.maximum(x - tau, 0.0) * oi_ref[...]


def kernel(midis_out):
    batch, two, n_insts, time = midis_out.shape
    assert time % _LST == 0

    bc = batch * two
    x3 = midis_out.reshape(bc, n_insts, time)

    T_BLK = 512
    out_inst = pl.pallas_call(
        _inst_kernel,
        grid=(bc, time // T_BLK),
        in_specs=[pl.BlockSpec((1, n_insts, T_BLK), lambda i, j: (i, 0, j))],
        out_specs=pl.BlockSpec((1, n_insts, T_BLK), lambda i, j: (i, 0, j)),
        out_shape=jax.ShapeDtypeStruct(x3.shape, x3.dtype),
    )(x3)

    rows = bc * n_insts * (time // _LST)
    x2 = midis_out.reshape(rows, _LST)
    oi2 = out_inst.reshape(rows, _LST)

    R_BLK = 1024
    out = pl.pallas_call(
        _time_kernel,
        grid=(rows // R_BLK,),
        in_specs=[
            pl.BlockSpec((R_BLK, _LST), lambda i: (i, 0)),
            pl.BlockSpec((R_BLK, _LST), lambda i: (i, 0)),
        ],
        out_specs=pl.BlockSpec((R_BLK, _LST), lambda i: (i, 0)),
        out_shape=jax.ShapeDtypeStruct(x2.shape, x2.dtype),
    )(x2, oi2)

    return out.reshape(batch, two, n_insts, time)


# speedup vs baseline: 3.8468x; 2.0764x over previous
import jax
import jax.numpy as jnp
from jax.experimental import pallas as pl

_LST = 64
_NEWTON_ITERS = 8


def _newton_tau(x, axis):
    tau = jnp.max(x, axis=axis, keepdims=True) - 1.0

    def body(_, tau):
        mask = (x > tau).astype(x.dtype)
        S = jnp.sum(x * mask, axis=axis, keepdims=True)
        C = jnp.sum(mask, axis=axis, keepdims=True)
        return jnp.where(C > 0.0, (S - 1.0) / jnp.maximum(C, 1.0), tau)

    return jax.lax.fori_loop(0, _NEWTON_ITERS, body, tau)


def _inst_kernel(x_ref, o_ref):
    x = x_ref[0]
    tau = _newton_tau(x, axis=0)
    o_ref[0] = jnp.maximum(x - tau, 0.0)


def _time_kernel(x_ref, oi_ref, o_ref):
    x = x_ref[...]
    tau = _newton_tau(x, axis=1)
    o_ref[...] = jnp.maximum(x - tau, 0.0) * oi_ref[...]


def kernel(midis_out):
    batch, two, n_insts, time = midis_out.shape
    assert time % _LST == 0

    bc = batch * two
    x3 = midis_out.reshape(bc, n_insts, time)

    T_BLK = 512
    out_inst = pl.pallas_call(
        _inst_kernel,
        grid=(bc, time // T_BLK),
        in_specs=[pl.BlockSpec((1, n_insts, T_BLK), lambda i, j: (i, 0, j))],
        out_specs=pl.BlockSpec((1, n_insts, T_BLK), lambda i, j: (i, 0, j)),
        out_shape=jax.ShapeDtypeStruct(x3.shape, x3.dtype),
    )(x3)

    rows = bc * n_insts * (time // _LST)
    x2 = midis_out.reshape(rows, _LST)
    oi2 = out_inst.reshape(rows, _LST)

    R_BLK = 1024
    out = pl.pallas_call(
        _time_kernel,
        grid=(rows // R_BLK,),
        in_specs=[
            pl.BlockSpec((R_BLK, _LST), lambda i: (i, 0)),
            pl.BlockSpec((R_BLK, _LST), lambda i: (i, 0)),
        ],
        out_specs=pl.BlockSpec((R_BLK, _LST), lambda i: (i, 0)),
        out_shape=jax.ShapeDtypeStruct(x2.shape, x2.dtype),
    )(x2, oi2)

    return out.reshape(batch, two, n_insts, time)


# speedup vs baseline: 5.4254x; 1.4103x over previous
import jax
import jax.numpy as jnp
from jax.experimental import pallas as pl

_LST = 64
_ITERS_INST = 8
_ITERS_TIME = 9


def _fused_kernel(x_ref, o_ref):
    x = x_ref[0]
    T = x.shape[1]
    nseg = T // _LST
    dt = x.dtype

    rM = jax.lax.broadcasted_iota(jnp.int32, (T, nseg), 0) // _LST
    cM = jax.lax.broadcasted_iota(jnp.int32, (T, nseg), 1)
    M = (rM == cM).astype(dt)
    rT = jax.lax.broadcasted_iota(jnp.int32, (nseg, T), 0)
    cT = jax.lax.broadcasted_iota(jnp.int32, (nseg, T), 1) // _LST
    Mt = (rT == cT).astype(dt)

    def dot(a, b):
        return jax.lax.dot(a, b, preferred_element_type=jnp.float32)

    tau_i = jnp.max(x, axis=0, keepdims=True) - 1.0

    def body_i(_, tau):
        mask = (x > tau).astype(dt)
        S = jnp.sum(x * mask, axis=0, keepdims=True)
        C = jnp.sum(mask, axis=0, keepdims=True)
        return jnp.where(C > 0.0, (S - 1.0) / jnp.maximum(C, 1.0), tau)

    tau_i = jax.lax.fori_loop(0, _ITERS_INST, body_i, tau_i)

    tau_t = (dot(x, M) - 1.0) / jnp.float32(_LST)

    def body_t(_, tau):
        tau_b = dot(tau, Mt)
        mask = (x > tau_b).astype(dt)
        S = dot(x * mask, M)
        C = dot(mask, M)
        return jnp.where(C > 0.0, (S - 1.0) / jnp.maximum(C, 1.0), tau)

    tau_t = jax.lax.fori_loop(0, _ITERS_TIME, body_t, tau_t)
    tau_tb = dot(tau_t, Mt)

    o_ref[0] = jnp.maximum(x - tau_i, 0.0) * jnp.maximum(x - tau_tb, 0.0)


def kernel(midis_out):
    batch, two, n_insts, time = midis_out.shape
    assert time % _LST == 0

    bc = batch * two
    x3 = midis_out.reshape(bc, n_insts, time)

    T_BLK = 512
    out = pl.pallas_call(
        _fused_kernel,
        grid=(bc, time // T_BLK),
        in_specs=[pl.BlockSpec((1, n_insts, T_BLK), lambda i, j: (i, 0, j))],
        out_specs=pl.BlockSpec((1, n_insts, T_BLK), lambda i, j: (i, 0, j)),
        out_shape=jax.ShapeDtypeStruct(x3.shape, x3.dtype),
    )(x3)

    return out.reshape(batch, two, n_insts, time)
